# baseline (speedup 1.0000x reference)
import functools

import jax
import jax.numpy as jnp
from jax import lax
from jax.experimental import pallas as pl
from jax.experimental.pallas import tpu as pltpu
from jax.experimental.pallas import tpu_sc as plsc

R, C = 128, 100000
L = 16
NW = 32
ROWS_PER_W = R // NW
UNROLL = 10
NCHUNK = C // (L * UNROLL)

_LN2 = 0.6931471805599453
_SQRT2 = 1.4142135623730951


def _vlog(s):
    bits = plsc.bitcast(s, jnp.int32)
    e = ((bits >> 23) & 0xFF) - 127
    m = plsc.bitcast((bits & 0x007FFFFF) | 0x3F800000, jnp.float32)
    big = m > _SQRT2
    m = jnp.where(big, m * 0.5, m)
    e = jnp.where(big, e + 1, e)
    t = (m - 1.0) / (m + 1.0)
    t2 = t * t
    p = 1.0 + t2 * (1.0 / 3.0 + t2 * (0.2 + t2 * (1.0 / 7.0)))
    return 2.0 * t * p + e.astype(jnp.float32) * _LN2


def _body(x_hbm, out_hbm, buf):
    wid = lax.axis_index("s") * 2 + lax.axis_index("c")
    for r in range(ROWS_PER_W):
        row = wid * ROWS_PER_W + r
        pltpu.sync_copy(x_hbm.at[row], buf)

        def maxstep(i, mv):
            base = i * (L * UNROLL)
            for u in range(UNROLL):
                mv = jnp.maximum(mv, buf[pl.ds(base + u * L, L)])
            return mv

        mv = lax.fori_loop(
            0, NCHUNK, maxstep, jnp.full((L,), -jnp.inf, jnp.float32))
        m = mv[0]
        for j in range(1, L):
            m = jnp.maximum(m, mv[j])
        mb = jnp.full((L,), m)

        def sumstep(i, sv):
            base = i * (L * UNROLL)
            for u in range(UNROLL):
                sv = sv + jnp.exp(buf[pl.ds(base + u * L, L)] - mb)
            return sv

        sv = lax.fori_loop(
            0, NCHUNK, sumstep, jnp.zeros((L,), jnp.float32))
        s = sv[0]
        for j in range(1, L):
            s = s + sv[j]

        lse = mb + _vlog(jnp.full((L,), s))

        def substep(i, c):
            base = i * (L * UNROLL)
            for u in range(UNROLL):
                sl = pl.ds(base + u * L, L)
                buf[sl] = buf[sl] - lse
            return c

        lax.fori_loop(0, NCHUNK, substep, 0)
        pltpu.sync_copy(buf, out_hbm.at[row])


@jax.jit
def kernel(x):
    mesh = plsc.VectorSubcoreMesh(core_axis_name="c", subcore_axis_name="s")
    return pl.kernel(
        _body,
        out_type=jax.ShapeDtypeStruct((R, C), jnp.float32),
        mesh=mesh,
        scratch_types=[pltpu.VMEM((C,), jnp.float32)],
        compiler_params=pltpu.CompilerParams(needs_layout_passes=False),
    )(x)


# speedup vs baseline: 1.0015x; 1.0015x over previous
import functools

import jax
import jax.numpy as jnp
from jax import lax
from jax.experimental import pallas as pl
from jax.experimental.pallas import tpu as pltpu
from jax.experimental.pallas import tpu_sc as plsc

R, C = 128, 100000
L = 16
NW = 32
ROWS_PER_W = R // NW
UNROLL = 10
NCHUNK = C // (L * UNROLL)

_LN2 = 0.6931471805599453
_SQRT2 = 1.4142135623730951


def _vlog(s):
    bits = plsc.bitcast(s, jnp.int32)
    e = ((bits >> 23) & 0xFF) - 127
    m = plsc.bitcast((bits & 0x007FFFFF) | 0x3F800000, jnp.float32)
    big = m > _SQRT2
    m = jnp.where(big, m * 0.5, m)
    e = jnp.where(big, e + 1, e)
    t = (m - 1.0) / (m + 1.0)
    t2 = t * t
    p = 1.0 + t2 * (1.0 / 3.0 + t2 * (0.2 + t2 * (1.0 / 7.0)))
    return 2.0 * t * p + e.astype(jnp.float32) * _LN2


def _body(x_hbm, out_hbm, buf):
    wid = lax.axis_index("s") * 2 + lax.axis_index("c")
    for r in range(ROWS_PER_W):
        row = wid * ROWS_PER_W + r
        with jax.named_scope("dma_in"):
            pltpu.sync_copy(x_hbm.at[row], buf)

        def maxstep(i, mv):
            base = i * (L * UNROLL)
            for u in range(UNROLL):
                mv = jnp.maximum(mv, buf[pl.ds(base + u * L, L)])
            return mv

        with jax.named_scope("pass_max"):
            mv = lax.fori_loop(
                0, NCHUNK, maxstep, jnp.full((L,), -jnp.inf, jnp.float32))
        m = mv[0]
        for j in range(1, L):
            m = jnp.maximum(m, mv[j])
        mb = jnp.full((L,), m)

        def sumstep(i, sv):
            base = i * (L * UNROLL)
            for u in range(UNROLL):
                sv = sv + jnp.exp(buf[pl.ds(base + u * L, L)] - mb)
            return sv

        with jax.named_scope("pass_sum"):
            sv = lax.fori_loop(
                0, NCHUNK, sumstep, jnp.zeros((L,), jnp.float32))
        s = sv[0]
        for j in range(1, L):
            s = s + sv[j]

        lse = mb + _vlog(jnp.full((L,), s))

        def substep(i, c):
            base = i * (L * UNROLL)
            for u in range(UNROLL):
                sl = pl.ds(base + u * L, L)
                buf[sl] = buf[sl] - lse
            return c

        with jax.named_scope("pass_sub"):
            lax.fori_loop(0, NCHUNK, substep, 0)
        with jax.named_scope("dma_out"):
            pltpu.sync_copy(buf, out_hbm.at[row])


@jax.jit
def kernel(x):
    mesh = plsc.VectorSubcoreMesh(core_axis_name="c", subcore_axis_name="s")
    return pl.kernel(
        _body,
        out_type=jax.ShapeDtypeStruct((R, C), jnp.float32),
        mesh=mesh,
        scratch_types=[pltpu.VMEM((C,), jnp.float32)],
        compiler_params=pltpu.CompilerParams(
            needs_layout_passes=False, use_tc_tiling_on_sc=True),
    )(x)


# speedup vs baseline: 1.7481x; 1.7455x over previous
import functools

import jax
import jax.numpy as jnp
from jax import lax
from jax.experimental import pallas as pl
from jax.experimental.pallas import tpu as pltpu
from jax.experimental.pallas import tpu_sc as plsc

R, C = 128, 100000
L = 16
G = R // L
NW = 32

S_SC = 25600
CB = 80
KTOT = S_SC // CB // NW
NBUF = 4

BT = 800
BT2 = 2000
NBLK_TC1 = (C - S_SC) // BT
NBLK_TC2 = C // BT2

_NEG_BIG = -3.0e38


def _k1_sc_body(xt_hbm, part_hbm, *refs):
    bufs = refs[:NBUF]
    isems = refs[NBUF:2 * NBUF]
    pbuf = refs[2 * NBUF]
    psem = refs[2 * NBUF + 1]
    w = lax.axis_index("s") * 2 + lax.axis_index("c")

    def start_in(k, b):
        ci = w + k * NW
        return pltpu.async_copy(
            xt_hbm.at[pl.ds(ci * CB, CB)], bufs[b], isems[b])

    ins = [start_in(k, k) for k in range(NBUF)]
    m = tuple(jnp.full((L,), _NEG_BIG, jnp.float32) for _ in range(G))
    s = tuple(jnp.zeros((L,), jnp.float32) for _ in range(G))
    for k in range(KTOT):
        b = k % NBUF
        ins[b].wait()
        buf = bufs[b]

        def maxstep(r, mg):
            return tuple(
                jnp.maximum(mg[g], buf[r, pl.ds(g * L, L)])
                for g in range(G))
        mc = lax.fori_loop(
            0, CB, maxstep,
            tuple(jnp.full((L,), _NEG_BIG, jnp.float32) for _ in range(G)))

        def sumstep(r, sg):
            return tuple(
                sg[g] + jnp.exp(buf[r, pl.ds(g * L, L)] - mc[g])
                for g in range(G))
        sc = lax.fori_loop(
            0, CB, sumstep,
            tuple(jnp.zeros((L,), jnp.float32) for _ in range(G)))

        if k + NBUF < KTOT:
            ins[b] = start_in(k + NBUF, b)
        mn, sn = [], []
        for g in range(G):
            hi = jnp.maximum(m[g], mc[g])
            sn.append(s[g] * jnp.exp(m[g] - hi)
                      + sc[g] * jnp.exp(mc[g] - hi))
            mn.append(hi)
        m, s = tuple(mn), tuple(sn)
    for g in range(G):
        pbuf[0, pl.ds(g * L, L)] = m[g]
        pbuf[1, pl.ds(g * L, L)] = s[g]
    pltpu.async_copy(pbuf, part_hbm.at[pl.ds(2 * w, 2)], psem).wait()


def _k1_tc_body(x_ref, p_ref):
    i = pl.program_id(0)
    blk = x_ref[...]
    bm = jnp.max(blk, axis=0, keepdims=True)
    bs = jnp.sum(jnp.exp(blk - bm), axis=0, keepdims=True)

    @pl.when(i == 0)
    def _():
        p_ref[0:1, :] = bm
        p_ref[1:2, :] = bs

    @pl.when(i > 0)
    def _():
        m = p_ref[0:1, :]
        s = p_ref[1:2, :]
        hi = jnp.maximum(m, bm)
        p_ref[1:2, :] = s * jnp.exp(m - hi) + bs * jnp.exp(bm - hi)
        p_ref[0:1, :] = hi


def _k2_tc_body(psc_ref, ptc_ref, x_ref, o_ref, lse_ref):
    i = pl.program_id(0)

    @pl.when(i == 0)
    def _():
        m = ptc_ref[0:1, :]
        s = ptc_ref[1:2, :]
        for w in range(NW):
            mw = psc_ref[2 * w:2 * w + 1, :]
            sw = psc_ref[2 * w + 1:2 * w + 2, :]
            hi = jnp.maximum(m, mw)
            s = s * jnp.exp(m - hi) + sw * jnp.exp(mw - hi)
            m = hi
        lse_ref[0:1, :] = m + jnp.log(s)

    o_ref[...] = x_ref[...] - lse_ref[0:1, :]


@jax.jit
def kernel(x):
    xt = x.T

    p_sc = pl.kernel(
        _k1_sc_body,
        out_type=jax.ShapeDtypeStruct((2 * NW, R), jnp.float32),
        mesh=plsc.VectorSubcoreMesh(
            core_axis_name="c", subcore_axis_name="s"),
        scratch_types=(
            [pltpu.VMEM((CB, R), jnp.float32) for _ in range(NBUF)]
            + [pltpu.SemaphoreType.DMA for _ in range(NBUF)]
            + [pltpu.VMEM((2, R), jnp.float32), pltpu.SemaphoreType.DMA]
        ),
        compiler_params=pltpu.CompilerParams(needs_layout_passes=False),
    )(xt)

    p_tc = pl.pallas_call(
        _k1_tc_body,
        grid=(NBLK_TC1,),
        in_specs=[pl.BlockSpec((BT, R), lambda i: (S_SC // BT + i, 0))],
        out_specs=pl.BlockSpec((8, R), lambda i: (0, 0)),
        out_shape=jax.ShapeDtypeStruct((8, R), jnp.float32),
    )(xt)

    out_t = pl.pallas_call(
        _k2_tc_body,
        grid=(NBLK_TC2,),
        in_specs=[
            pl.BlockSpec((2 * NW, R), lambda i: (0, 0)),
            pl.BlockSpec((8, R), lambda i: (0, 0)),
            pl.BlockSpec((BT2, R), lambda i: (i, 0)),
        ],
        out_specs=pl.BlockSpec((BT2, R), lambda i: (i, 0)),
        out_shape=jax.ShapeDtypeStruct((C, R), jnp.float32),
        scratch_shapes=[pltpu.VMEM((8, R), jnp.float32)],
    )(p_sc, p_tc, xt)
    return out_t.T


# speedup vs baseline: 2.2014x; 1.2593x over previous
import jax
import jax.numpy as jnp
from jax import lax
from jax.experimental import pallas as pl
from jax.experimental.pallas import tpu as pltpu
from jax.experimental.pallas import tpu_sc as plsc

R, C = 128, 100000
L = 16
G = R // L
NW = 32

S_SC = 32000
CB = 200
KTOT = S_SC // CB // NW
NBUF = 4

BT2 = 2000
NV = (C - S_SC) // BT2
NBLK2 = C // BT2

_NEG_BIG = -3.0e38


def _k_sc_body(xt_hbm, part_hbm, *refs):
    bufs = refs[:NBUF]
    isems = refs[NBUF:2 * NBUF]
    pbuf = refs[2 * NBUF]
    psem = refs[2 * NBUF + 1]
    w = lax.axis_index("s") * 2 + lax.axis_index("c")

    def start_in(k, b):
        ci = w + k * NW
        return pltpu.async_copy(
            xt_hbm.at[pl.ds(C - S_SC + ci * CB, CB)], bufs[b], isems[b])

    ins = [start_in(k, k) for k in range(NBUF)]
    m = tuple(jnp.full((L,), _NEG_BIG, jnp.float32) for _ in range(G))
    s = tuple(jnp.zeros((L,), jnp.float32) for _ in range(G))
    for k in range(KTOT):
        b = k % NBUF
        ins[b].wait()
        buf = bufs[b]

        def maxstep(r, mg):
            return tuple(
                jnp.maximum(mg[g], buf[r, pl.ds(g * L, L)])
                for g in range(G))
        mc = lax.fori_loop(
            0, CB, maxstep,
            tuple(jnp.full((L,), _NEG_BIG, jnp.float32) for _ in range(G)))

        def sumstep(r, sg):
            return tuple(
                sg[g] + jnp.exp(buf[r, pl.ds(g * L, L)] - mc[g])
                for g in range(G))
        sc = lax.fori_loop(
            0, CB, sumstep,
            tuple(jnp.zeros((L,), jnp.float32) for _ in range(G)))

        if k + NBUF < KTOT:
            ins[b] = start_in(k + NBUF, b)
        mn, sn = [], []
        for g in range(G):
            hi = jnp.maximum(m[g], mc[g])
            sn.append(s[g] * jnp.exp(m[g] - hi)
                      + sc[g] * jnp.exp(mc[g] - hi))
            mn.append(hi)
        m, s = tuple(mn), tuple(sn)
    for g in range(G):
        pbuf[0, pl.ds(g * L, L)] = m[g]
        pbuf[1, pl.ds(g * L, L)] = s[g]
    pltpu.async_copy(pbuf, part_hbm.at[pl.ds(2 * w, 2)], psem).wait()


def _k_tc_body(psc_ref, x_ref, o_ref, big_ref, acc_ref, lse_ref):
    i = pl.program_id(0)

    @pl.when(i < NV)
    def _():
        blk = x_ref[...]
        big_ref[pl.ds(i * BT2, BT2), :] = blk
        bm = jnp.max(blk, axis=0, keepdims=True)
        bs = jnp.sum(jnp.exp(blk - bm), axis=0, keepdims=True)

        @pl.when(i == 0)
        def _():
            acc_ref[0:1, :] = bm
            acc_ref[1:2, :] = bs

        @pl.when(i > 0)
        def _():
            m = acc_ref[0:1, :]
            s = acc_ref[1:2, :]
            hi = jnp.maximum(m, bm)
            acc_ref[1:2, :] = s * jnp.exp(m - hi) + bs * jnp.exp(bm - hi)
            acc_ref[0:1, :] = hi

    @pl.when(i == NV)
    def _():
        m = acc_ref[0:1, :]
        s = acc_ref[1:2, :]
        for w in range(NW):
            mw = psc_ref[2 * w:2 * w + 1, :]
            sw = psc_ref[2 * w + 1:2 * w + 2, :]
            hi = jnp.maximum(m, mw)
            s = s * jnp.exp(m - hi) + sw * jnp.exp(mw - hi)
            m = hi
        lse_ref[0:1, :] = m + jnp.log(s)

    @pl.when(i >= NV)
    def _():
        j = i - NV
        lse = lse_ref[0:1, :]

        @pl.when(j < NV)
        def _():
            o_ref[...] = big_ref[pl.ds(j * BT2, BT2), :] - lse

        @pl.when(j >= NV)
        def _():
            o_ref[...] = x_ref[...] - lse


def _x_idx(i):
    j = i - NV
    return (jnp.where(i < NV, i, jnp.where(j < NV, NV - 1, j)), 0)


def _o_idx(i):
    return (jnp.where(i < NV, 0, i - NV), 0)


@jax.jit
def kernel(x):
    xt = x.T

    p_sc = pl.kernel(
        _k_sc_body,
        out_type=jax.ShapeDtypeStruct((2 * NW, R), jnp.float32),
        mesh=plsc.VectorSubcoreMesh(
            core_axis_name="c", subcore_axis_name="s"),
        scratch_types=(
            [pltpu.VMEM((CB, R), jnp.float32) for _ in range(NBUF)]
            + [pltpu.SemaphoreType.DMA for _ in range(NBUF)]
            + [pltpu.VMEM((2, R), jnp.float32), pltpu.SemaphoreType.DMA]
        ),
        compiler_params=pltpu.CompilerParams(needs_layout_passes=False),
    )(xt)

    out_t = pl.pallas_call(
        _k_tc_body,
        grid=(NV + NBLK2,),
        in_specs=[
            pl.BlockSpec((2 * NW, R), lambda i: (0, 0)),
            pl.BlockSpec((BT2, R), _x_idx),
        ],
        out_specs=pl.BlockSpec((BT2, R), _o_idx),
        out_shape=jax.ShapeDtypeStruct((C, R), jnp.float32),
        scratch_shapes=[
            pltpu.VMEM((NV * BT2, R), jnp.float32),
            pltpu.VMEM((8, R), jnp.float32),
            pltpu.VMEM((8, R), jnp.float32),
        ],
    )(p_sc, xt)
    return out_t.T


# speedup vs baseline: 2.3044x; 1.0468x over previous
import jax
import jax.numpy as jnp
from jax import lax
from jax.experimental import pallas as pl
from jax.experimental.pallas import tpu as pltpu
from jax.experimental.pallas import tpu_sc as plsc

R, C = 128, 100000
L = 16
G = R // L
NW = 32

S_SC = 32000
CB = 200
KTOT = S_SC // CB // NW
NBUF = 4

BT2 = 2000
NV = (C - S_SC) // BT2
NBLK2 = C // BT2

_NEG_BIG = -3.0e38


def _k_sc_body(xt_hbm, part_hbm, *refs):
    bufs = refs[:NBUF]
    isems = refs[NBUF:2 * NBUF]
    pbuf = refs[2 * NBUF]
    psem = refs[2 * NBUF + 1]
    w = lax.axis_index("s") * 2 + lax.axis_index("c")

    def start_in(k, b):
        ci = w + k * NW
        return pltpu.async_copy(
            xt_hbm.at[pl.ds(C - S_SC + ci * CB, CB)], bufs[b], isems[b])

    ins = [start_in(k, k) for k in range(NBUF)]
    m = tuple(jnp.full((L,), _NEG_BIG, jnp.float32) for _ in range(G))
    s = tuple(jnp.zeros((L,), jnp.float32) for _ in range(G))
    for k in range(KTOT):
        b = k % NBUF
        ins[b].wait()
        buf = bufs[b]

        def maxstep(r, mg):
            return tuple(
                jnp.maximum(mg[g], buf[r, pl.ds(g * L, L)])
                for g in range(G))
        mc = lax.fori_loop(
            0, CB, maxstep,
            tuple(jnp.full((L,), _NEG_BIG, jnp.float32) for _ in range(G)))

        def sumstep(r, sg):
            return tuple(
                sg[g] + jnp.exp(buf[r, pl.ds(g * L, L)] - mc[g])
                for g in range(G))
        sc = lax.fori_loop(
            0, CB, sumstep,
            tuple(jnp.zeros((L,), jnp.float32) for _ in range(G)))

        if k + NBUF < KTOT:
            ins[b] = start_in(k + NBUF, b)
        mn, sn = [], []
        for g in range(G):
            hi = jnp.maximum(m[g], mc[g])
            sn.append(s[g] * jnp.exp(m[g] - hi)
                      + sc[g] * jnp.exp(mc[g] - hi))
            mn.append(hi)
        m, s = tuple(mn), tuple(sn)
    for g in range(G):
        pbuf[0, pl.ds(g * L, L)] = m[g]
        pbuf[1, pl.ds(g * L, L)] = s[g]
    pltpu.async_copy(pbuf, part_hbm.at[pl.ds(2 * w, 2)], psem).wait()


def _k_tc_body(psc_ref, x_ref, o_ref, big_ref, acc_ref, lse_ref):
    i = pl.program_id(0)

    @pl.when(i < NV)
    def _():
        blk = x_ref[...]
        big_ref[pl.ds(i * BT2, BT2), :] = blk
        bm = jnp.max(blk, axis=0, keepdims=True)
        bs = jnp.sum(jnp.exp(blk - bm), axis=0, keepdims=True)

        @pl.when(i == 0)
        def _():
            acc_ref[0:1, :] = bm
            acc_ref[1:2, :] = bs

        @pl.when(i > 0)
        def _():
            m = acc_ref[0:1, :]
            s = acc_ref[1:2, :]
            hi = jnp.maximum(m, bm)
            acc_ref[1:2, :] = s * jnp.exp(m - hi) + bs * jnp.exp(bm - hi)
            acc_ref[0:1, :] = hi

    @pl.when(i == NV)
    def _():
        m = acc_ref[0:1, :]
        s = acc_ref[1:2, :]
        for w in range(NW):
            mw = psc_ref[2 * w:2 * w + 1, :]
            sw = psc_ref[2 * w + 1:2 * w + 2, :]
            hi = jnp.maximum(m, mw)
            s = s * jnp.exp(m - hi) + sw * jnp.exp(mw - hi)
            m = hi
        lse_ref[0:1, :] = m + jnp.log(s)

    @pl.when(i >= NV)
    def _():
        j = i - NV
        lse = lse_ref[0:1, :]

        @pl.when(j < NV)
        def _():
            o_ref[...] = big_ref[pl.ds(j * BT2, BT2), :] - lse

        @pl.when(j >= NV)
        def _():
            o_ref[...] = x_ref[...] - lse


def _x_idx(i):
    j = i - NV
    return (jnp.where(i < NV, i, jnp.where(j < NV, NV - 1, j)), 0)


def _o_idx(i):
    return (jnp.where(i < NV, 0, i - NV), 0)


@jax.jit
def kernel(x):
    xt = x.T

    p_sc = pl.kernel(
        _k_sc_body,
        out_type=jax.ShapeDtypeStruct((2 * NW, R), jnp.float32),
        mesh=plsc.VectorSubcoreMesh(
            core_axis_name="c", subcore_axis_name="s"),
        scratch_types=(
            [pltpu.VMEM((CB, R), jnp.float32) for _ in range(NBUF)]
            + [pltpu.SemaphoreType.DMA for _ in range(NBUF)]
            + [pltpu.VMEM((2, R), jnp.float32), pltpu.SemaphoreType.DMA]
        ),
        compiler_params=pltpu.CompilerParams(
            needs_layout_passes=False, use_tc_tiling_on_sc=True),
    )(xt)

    out_t = pl.pallas_call(
        _k_tc_body,
        grid=(NV + NBLK2,),
        in_specs=[
            pl.BlockSpec((2 * NW, R), lambda i: (0, 0)),
            pl.BlockSpec((BT2, R), _x_idx),
        ],
        out_specs=pl.BlockSpec((BT2, R), _o_idx),
        out_shape=jax.ShapeDtypeStruct((C, R), jnp.float32),
        scratch_shapes=[
            pltpu.VMEM((NV * BT2, R), jnp.float32),
            pltpu.VMEM((8, R), jnp.float32),
            pltpu.VMEM((8, R), jnp.float32),
        ],
    )(p_sc, xt)
    return out_t.T
